# baseline (speedup 1.0000x reference)
import functools

import jax
import jax.numpy as jnp
from jax import lax
from jax.experimental import pallas as pl
from jax.experimental.pallas import tpu as pltpu
from jax.experimental.pallas import tpu_sc as plsc

N = 10000
NP = 10240
E = 320000
D = 128
DE = 144
NEG = 0.2
BLK = 1024
GRID = NP // BLK
NC = 2
NS = 16
NW = NC * NS
EPW = E // NW
K = 80
NCH = EPW // K
NBUF = 3
NIDX = 4
SPAN = 12
ROWS_PER_TILE = NP // NS

_F32 = jnp.float32
_HI = jax.lax.Precision.HIGHEST


def _proj_tail(fb, al_ref, ar_ref, feat_ref, er_ref, shift_ref,
               mel_ref, mer_ref):
    el = jnp.sum(fb * al_ref[...], axis=1, keepdims=True)
    er = jnp.sum(fb * ar_ref[...], axis=1, keepdims=True)
    lane = lax.broadcasted_iota(jnp.int32, (BLK, DE - D), 1)
    pad = jnp.where(lane == 0, 1.0, jnp.where(lane == 1, el, 0.0)).astype(_F32)
    feat_ref[...] = jnp.concatenate([fb, pad], axis=1)
    er_ref[...] = er.reshape(1, BLK, 1)

    i = pl.program_id(0)

    @pl.when(i == 0)
    def _init():
        mel_ref[0] = jnp.float32(-3.0e38)
        mer_ref[0] = jnp.float32(-3.0e38)

    mel_ref[0] = jnp.maximum(mel_ref[0], jnp.max(el))
    mer_ref[0] = jnp.maximum(mer_ref[0], jnp.max(er))

    @pl.when(i == GRID - 1)
    def _emit():
        m = mel_ref[0] + mer_ref[0]
        m = jnp.where(m > 0.0, m, NEG * m)
        shift_ref[...] = jnp.full((1, 1, 16), m, _F32)


def _proj_body(x_ref, w_ref, al_ref, ar_ref, feat_ref, er_ref,
               shift_ref, mel_ref, mer_ref):
    fb = jnp.dot(x_ref[...], w_ref[...], precision=_HI)
    _proj_tail(fb, al_ref, ar_ref, feat_ref, er_ref, shift_ref,
               mel_ref, mer_ref)


def _merge_norm(a0_ref, a1_ref, b_ref):
    r = a0_ref[...] + a1_ref[...]
    den = r[:, D:D + 1]
    den = jnp.where(den > 0.0, den, 1.0)
    return r[:, :D] / den + b_ref[...]


def _norm_proj_body(a0_ref, a1_ref, b_ref, w_ref, al_ref, ar_ref,
                    feat_ref, er_ref, shift_ref, mel_ref, mer_ref):
    h = _merge_norm(a0_ref, a1_ref, b_ref)
    h = jnp.where(h > 0.0, h, jnp.exp(jnp.minimum(h, 0.0)) - 1.0)
    fb = jnp.dot(h, w_ref[...], precision=_HI)
    _proj_tail(fb, al_ref, ar_ref, feat_ref, er_ref, shift_ref,
               mel_ref, mer_ref)


def _final_body(a0_ref, a1_ref, b_ref, out_ref):
    out_ref[...] = _merge_norm(a0_ref, a1_ref, b_ref)


_PROJ_OUTS = (
    jax.ShapeDtypeStruct((NP, DE), _F32),
    jax.ShapeDtypeStruct((GRID, BLK, 1), _F32),
    jax.ShapeDtypeStruct((1, 1, 16), _F32),
)
_PROJ_OUT_SPECS = [
    pl.BlockSpec((BLK, DE), lambda i: (i, 0)),
    pl.BlockSpec((1, BLK, 1), lambda i: (i, 0, 0)),
    pl.BlockSpec((1, 1, 16), lambda i: (0, 0, 0)),
]
_PROJ_SCRATCH = [pltpu.SMEM((1,), _F32), pltpu.SMEM((1,), _F32)]
_FULL2 = pl.BlockSpec((D, D), lambda i: (0, 0))
_ROW = pl.BlockSpec((1, D), lambda i: (0, 0))
_ACC0 = pl.BlockSpec((BLK, DE), lambda i: (i, 0))
_ACC1 = pl.BlockSpec((BLK, DE), lambda i: (i + GRID, 0))


def _proj(x, w, al, ar):
    return pl.pallas_call(
        _proj_body,
        grid=(GRID,),
        in_specs=[pl.BlockSpec((BLK, D), lambda i: (i, 0)), _FULL2, _ROW, _ROW],
        out_specs=_PROJ_OUT_SPECS,
        out_shape=_PROJ_OUTS,
        scratch_shapes=_PROJ_SCRATCH,
    )(x, w, al, ar)


def _norm_proj(acc, b, w, al, ar):
    return pl.pallas_call(
        _norm_proj_body,
        grid=(GRID,),
        in_specs=[_ACC0, _ACC1, _ROW, _FULL2, _ROW, _ROW],
        out_specs=_PROJ_OUT_SPECS,
        out_shape=_PROJ_OUTS,
        scratch_shapes=_PROJ_SCRATCH,
    )(acc, acc, b, w, al, ar)


def _final(acc, b):
    return pl.pallas_call(
        _final_body,
        grid=(GRID,),
        in_specs=[_ACC0, _ACC1, _ROW],
        out_specs=pl.BlockSpec((BLK, D), lambda i: (i, 0)),
        out_shape=jax.ShapeDtypeStruct((NP, D), _F32),
    )(acc, acc, b)


@functools.partial(
    pl.kernel,
    out_type=jax.ShapeDtypeStruct((NC * NP, DE), _F32),
    mesh=plsc.VectorSubcoreMesh(core_axis_name="c", subcore_axis_name="s"),
    compiler_params=pltpu.CompilerParams(
        needs_layout_passes=False, use_tc_tiling_on_sc=False),
    scratch_types=(
        [pltpu.VMEM((16,), _F32),
         pltpu.VMEM((K,), _F32)]
        + [pltpu.VMEM((K,), jnp.int32),
           pltpu.VMEM((K,), jnp.int32),
           pltpu.SemaphoreType.DMA] * NIDX
        + [pltpu.VMEM((K, DE), _F32),
           pltpu.VMEM((K,), _F32),
           pltpu.SemaphoreType.DMA,
           pltpu.SemaphoreType.DMA] * NBUF
        + [pltpu.VMEM_SHARED((NP, DE), _F32)]
    ),
)
def _sc_edge(feat_hbm, er_hbm, shift_hbm, src_hbm, dst_hbm, out_hbm,
             shift_v, ee_v, *rest):
    idxs = [rest[3 * i:3 * i + 3] for i in range(NIDX)]
    rows = [rest[3 * NIDX + 4 * b:3 * NIDX + 4 * b + 4] for b in range(NBUF)]
    acc_sh = rest[3 * NIDX + 4 * NBUF]
    c = lax.axis_index("c")
    s = lax.axis_index("s")
    wid = c * NS + s
    buf0 = rows[0][0]

    def _zrow(i, carry):
        for j in range(DE // 16):
            buf0[i, pl.ds(j * 16, 16)] = jnp.zeros((16,), _F32)
        return carry

    lax.fori_loop(0, K, _zrow, 0)
    for t in range(ROWS_PER_TILE // K):
        pltpu.sync_copy(buf0, acc_sh.at[pl.ds(s * ROWS_PER_TILE + t * K, K)])

    pltpu.sync_copy(shift_hbm, shift_v)
    plsc.subcore_barrier()
    shift = shift_v[...]

    def _issue_idx(jj, i):
        base = wid * EPW + jj * K
        src_v, dst_v, isem = idxs[i]
        pltpu.async_copy(src_hbm.at[pl.ds(base, K)], src_v, isem)
        pltpu.async_copy(dst_hbm.at[pl.ds(base, K)], dst_v, isem)

    def _wait_idx(i):
        src_v, dst_v, isem = idxs[i]
        pltpu.make_async_copy(src_hbm.at[pl.ds(0, K)], src_v, isem).wait()
        pltpu.make_async_copy(dst_hbm.at[pl.ds(0, K)], dst_v, isem).wait()

    def _issue_gather(b, i):
        buf, erg, gsem, _ = rows[b]
        pltpu.async_copy(feat_hbm.at[idxs[i][0]], buf, gsem)
        pltpu.async_copy(er_hbm.at[idxs[i][1]], erg, gsem)

    def _wait_gather(b, i):
        buf, erg, gsem, _ = rows[b]
        pltpu.make_async_copy(feat_hbm.at[idxs[i][0]], buf, gsem).wait()
        pltpu.make_async_copy(er_hbm.at[idxs[i][1]], erg, gsem).wait()

    def _issue_scatter(b, i):
        buf, _, _, ssem = rows[b]
        pltpu.async_copy(buf, acc_sh.at[idxs[i][1]], ssem, add=True)

    def _wait_scatter(b, i):
        buf, _, _, ssem = rows[b]
        pltpu.make_async_copy(buf, acc_sh.at[idxs[i][1]], ssem).wait()

    def _process(b):
        buf, erg, _, _ = rows[b]

        @plsc.parallel_loop(0, K // 16, unroll=2)
        def _ee(i):
            grp = lax.iota(jnp.int32, 16) + i * 16
            elg = plsc.load_gather(buf, [grp, jnp.full((16,), D + 1, jnp.int32)])
            e = elg + erg[pl.ds(i * 16, 16)]
            e = jnp.where(e > 0.0, e, NEG * e) - shift
            ee16 = jnp.exp(e)
            ee_v[pl.ds(i * 16, 16)] = ee16
            plsc.store_scatter(buf, [grp, jnp.full((16,), D, jnp.int32)], ee16)

        @plsc.parallel_loop(0, K // 16, unroll=1)
        def _row(g):
            ee16 = ee_v[pl.ds(g * 16, 16)]
            for r in range(16):
                i = g * 16 + r
                sc = ee16.at[jnp.full((16,), r, jnp.int32)].get(
                    mode="promise_in_bounds")
                for j in range(D // 16):
                    buf[i, pl.ds(j * 16, 16)] = buf[i, pl.ds(j * 16, 16)] * sc

    def _step(jj, k):
        b = k % NBUF
        b1 = (k + 1) % NBUF
        i = k % NIDX
        i1 = (k + 1) % NIDX
        i2 = (k + 2) % NIDX

        @pl.when(jj >= 2)
        def _retire():
            _wait_scatter(b1, i2)

        @pl.when(jj + 1 < NCH)
        def _pref_gather():
            _wait_idx(i1)
            _issue_gather(b1, i1)

        _wait_gather(b, i)
        _process(b)
        _issue_scatter(b, i)

        @pl.when(jj + 2 < NCH)
        def _pref_idx():
            _issue_idx(jj + 2, i2)

    _issue_idx(0, 0)
    _issue_idx(1, 1)
    _wait_idx(0)
    _issue_gather(0, 0)

    def _span(p, carry):
        for k in range(SPAN):
            jj = p * SPAN + k

            @pl.when(jj < NCH)
            def _guarded():
                _step(jj, k)

        return carry

    lax.fori_loop(0, (NCH + SPAN - 1) // SPAN, _span, 0)
    _wait_scatter((NCH - 2) % NBUF, (NCH - 2) % NIDX)
    _wait_scatter((NCH - 1) % NBUF, (NCH - 1) % NIDX)
    plsc.subcore_barrier()

    r0 = s * ROWS_PER_TILE
    pltpu.sync_copy(acc_sh.at[pl.ds(r0, ROWS_PER_TILE)],
                    out_hbm.at[pl.ds(c * NP + r0, ROWS_PER_TILE)])


def kernel(x, edge_index, W1, al1, ar1, b1, W2, al2, ar2, b2):
    src = edge_index[0]
    dst = edge_index[1]
    xp = jnp.pad(x, ((0, NP - N), (0, 0)))
    feat1, er1, sh1 = _proj(xp, W1, al1, ar1)
    acc1 = _sc_edge(feat1, er1.reshape(NP), sh1.reshape(16), src, dst)
    feat2, er2, sh2 = _norm_proj(acc1, b1.reshape(1, D), W2, al2, ar2)
    acc2 = _sc_edge(feat2, er2.reshape(NP), sh2.reshape(16), src, dst)
    out = _final(acc2, b2.reshape(1, D))
    return out[:N]


# speedup vs baseline: 1.1774x; 1.1774x over previous
import functools

import jax
import jax.numpy as jnp
from jax import lax
from jax.experimental import pallas as pl
from jax.experimental.pallas import tpu as pltpu
from jax.experimental.pallas import tpu_sc as plsc

N = 10000
NP = 10240
E = 320000
D = 128
DE = 132
NEG = 0.2
BLK = 1024
GRID = NP // BLK
NC = 2
NS = 16
NW = NC * NS
EPW = E // NW
K = 80
NCH = EPW // K
NBUF = 3
NIDX = 4
SPAN = 12
ROWS_PER_TILE = NP // NS

_F32 = jnp.float32
_HI = jax.lax.Precision.HIGHEST


def _proj_tail(fb, al_ref, ar_ref, feat_ref, er_ref, shift_ref,
               mel_ref, mer_ref):
    el = jnp.sum(fb * al_ref[...], axis=1, keepdims=True)
    er = jnp.sum(fb * ar_ref[...], axis=1, keepdims=True)
    lane = lax.broadcasted_iota(jnp.int32, (BLK, DE - D), 1)
    pad = jnp.where(lane == 0, 1.0, jnp.where(lane == 1, el, 0.0)).astype(_F32)
    feat_ref[...] = jnp.concatenate([fb, pad], axis=1)
    er_ref[...] = er.reshape(1, BLK, 1)

    i = pl.program_id(0)

    @pl.when(i == 0)
    def _init():
        mel_ref[0] = jnp.float32(-3.0e38)
        mer_ref[0] = jnp.float32(-3.0e38)

    mel_ref[0] = jnp.maximum(mel_ref[0], jnp.max(el))
    mer_ref[0] = jnp.maximum(mer_ref[0], jnp.max(er))

    @pl.when(i == GRID - 1)
    def _emit():
        m = mel_ref[0] + mer_ref[0]
        m = jnp.where(m > 0.0, m, NEG * m)
        shift_ref[...] = jnp.full((1, 1, 16), m, _F32)


def _proj_body(x_ref, w_ref, al_ref, ar_ref, feat_ref, er_ref,
               shift_ref, mel_ref, mer_ref):
    fb = jnp.dot(x_ref[...], w_ref[...], precision=_HI)
    _proj_tail(fb, al_ref, ar_ref, feat_ref, er_ref, shift_ref,
               mel_ref, mer_ref)


def _merge_norm(a0_ref, a1_ref, b_ref):
    r = a0_ref[...] + a1_ref[...]
    den = r[:, D:D + 1]
    den = jnp.where(den > 0.0, den, 1.0)
    return r[:, :D] / den + b_ref[...]


def _norm_proj_body(a0_ref, a1_ref, b_ref, w_ref, al_ref, ar_ref,
                    feat_ref, er_ref, shift_ref, mel_ref, mer_ref):
    h = _merge_norm(a0_ref, a1_ref, b_ref)
    h = jnp.where(h > 0.0, h, jnp.exp(jnp.minimum(h, 0.0)) - 1.0)
    fb = jnp.dot(h, w_ref[...], precision=_HI)
    _proj_tail(fb, al_ref, ar_ref, feat_ref, er_ref, shift_ref,
               mel_ref, mer_ref)


def _final_body(a0_ref, a1_ref, b_ref, out_ref):
    out_ref[...] = _merge_norm(a0_ref, a1_ref, b_ref)


_PROJ_OUTS = (
    jax.ShapeDtypeStruct((NP, DE), _F32),
    jax.ShapeDtypeStruct((GRID, BLK, 1), _F32),
    jax.ShapeDtypeStruct((1, 1, 16), _F32),
)
_PROJ_OUT_SPECS = [
    pl.BlockSpec((BLK, DE), lambda i: (i, 0)),
    pl.BlockSpec((1, BLK, 1), lambda i: (i, 0, 0)),
    pl.BlockSpec((1, 1, 16), lambda i: (0, 0, 0)),
]
_PROJ_SCRATCH = [pltpu.SMEM((1,), _F32), pltpu.SMEM((1,), _F32)]
_FULL2 = pl.BlockSpec((D, D), lambda i: (0, 0))
_ROW = pl.BlockSpec((1, D), lambda i: (0, 0))
_ACC0 = pl.BlockSpec((BLK, DE), lambda i: (i, 0))
_ACC1 = pl.BlockSpec((BLK, DE), lambda i: (i + GRID, 0))


def _proj(x, w, al, ar):
    return pl.pallas_call(
        _proj_body,
        grid=(GRID,),
        in_specs=[pl.BlockSpec((BLK, D), lambda i: (i, 0)), _FULL2, _ROW, _ROW],
        out_specs=_PROJ_OUT_SPECS,
        out_shape=_PROJ_OUTS,
        scratch_shapes=_PROJ_SCRATCH,
    )(x, w, al, ar)


def _norm_proj(acc, b, w, al, ar):
    return pl.pallas_call(
        _norm_proj_body,
        grid=(GRID,),
        in_specs=[_ACC0, _ACC1, _ROW, _FULL2, _ROW, _ROW],
        out_specs=_PROJ_OUT_SPECS,
        out_shape=_PROJ_OUTS,
        scratch_shapes=_PROJ_SCRATCH,
    )(acc, acc, b, w, al, ar)


def _final(acc, b):
    return pl.pallas_call(
        _final_body,
        grid=(GRID,),
        in_specs=[_ACC0, _ACC1, _ROW],
        out_specs=pl.BlockSpec((BLK, D), lambda i: (i, 0)),
        out_shape=jax.ShapeDtypeStruct((NP, D), _F32),
    )(acc, acc, b)


@functools.partial(
    pl.kernel,
    out_type=jax.ShapeDtypeStruct((NC * NP, DE), _F32),
    mesh=plsc.VectorSubcoreMesh(core_axis_name="c", subcore_axis_name="s"),
    compiler_params=pltpu.CompilerParams(
        needs_layout_passes=False, use_tc_tiling_on_sc=False),
    scratch_types=(
        [pltpu.VMEM((16,), _F32),
         pltpu.VMEM((K,), _F32)]
        + [pltpu.VMEM((K,), jnp.int32),
           pltpu.VMEM((K,), jnp.int32),
           pltpu.SemaphoreType.DMA] * NIDX
        + [pltpu.VMEM((K, DE), _F32),
           pltpu.VMEM((K,), _F32),
           pltpu.SemaphoreType.DMA,
           pltpu.SemaphoreType.DMA] * NBUF
        + [pltpu.VMEM_SHARED((NP, DE), _F32)]
    ),
)
def _sc_edge(feat_hbm, er_hbm, shift_hbm, src_hbm, dst_hbm, out_hbm,
             shift_v, ee_v, *rest):
    idxs = [rest[3 * i:3 * i + 3] for i in range(NIDX)]
    rows = [rest[3 * NIDX + 4 * b:3 * NIDX + 4 * b + 4] for b in range(NBUF)]
    acc_sh = rest[3 * NIDX + 4 * NBUF]
    c = lax.axis_index("c")
    s = lax.axis_index("s")
    wid = c * NS + s
    buf0 = rows[0][0]

    @plsc.parallel_loop(0, K, unroll=4)
    def _zrow(i):
        for j in range(D // 16):
            buf0[i, pl.ds(j * 16, 16)] = jnp.zeros((16,), _F32)

    @plsc.parallel_loop(0, K // 16, unroll=1)
    def _zden(g):
        grp = lax.iota(jnp.int32, 16) + g * 16
        plsc.store_scatter(buf0, [grp, jnp.full((16,), D, jnp.int32)],
                           jnp.zeros((16,), _F32))
        plsc.store_scatter(buf0, [grp, jnp.full((16,), D + 1, jnp.int32)],
                           jnp.zeros((16,), _F32))
        plsc.store_scatter(buf0, [grp, jnp.full((16,), D + 2, jnp.int32)],
                           jnp.zeros((16,), _F32))
        plsc.store_scatter(buf0, [grp, jnp.full((16,), D + 3, jnp.int32)],
                           jnp.zeros((16,), _F32))
    for t in range(ROWS_PER_TILE // K):
        pltpu.sync_copy(buf0, acc_sh.at[pl.ds(s * ROWS_PER_TILE + t * K, K)])

    pltpu.sync_copy(shift_hbm, shift_v)
    plsc.subcore_barrier()
    shift = shift_v[...]

    def _issue_idx(jj, i):
        base = wid * EPW + jj * K
        src_v, dst_v, isem = idxs[i]
        pltpu.async_copy(src_hbm.at[pl.ds(base, K)], src_v, isem)
        pltpu.async_copy(dst_hbm.at[pl.ds(base, K)], dst_v, isem)

    def _wait_idx(i):
        src_v, dst_v, isem = idxs[i]
        pltpu.make_async_copy(src_hbm.at[pl.ds(0, K)], src_v, isem).wait()
        pltpu.make_async_copy(dst_hbm.at[pl.ds(0, K)], dst_v, isem).wait()

    def _issue_gather(b, i):
        buf, erg, gsem, _ = rows[b]
        pltpu.async_copy(feat_hbm.at[idxs[i][0]], buf, gsem)
        pltpu.async_copy(er_hbm.at[idxs[i][1]], erg, gsem)

    def _wait_gather(b, i):
        buf, erg, gsem, _ = rows[b]
        pltpu.make_async_copy(feat_hbm.at[idxs[i][0]], buf, gsem).wait()
        pltpu.make_async_copy(er_hbm.at[idxs[i][1]], erg, gsem).wait()

    def _issue_scatter(b, i):
        buf, _, _, ssem = rows[b]
        pltpu.async_copy(buf, acc_sh.at[idxs[i][1]], ssem, add=True)

    def _wait_scatter(b, i):
        buf, _, _, ssem = rows[b]
        pltpu.make_async_copy(buf, acc_sh.at[idxs[i][1]], ssem).wait()

    def _process(b):
        buf, erg, _, _ = rows[b]

        @plsc.parallel_loop(0, K // 16, unroll=2)
        def _ee(i):
            grp = lax.iota(jnp.int32, 16) + i * 16
            elg = plsc.load_gather(buf, [grp, jnp.full((16,), D + 1, jnp.int32)])
            e = elg + erg[pl.ds(i * 16, 16)]
            e = jnp.where(e > 0.0, e, NEG * e) - shift
            ee16 = jnp.exp(e)
            ee_v[pl.ds(i * 16, 16)] = ee16
            plsc.store_scatter(buf, [grp, jnp.full((16,), D, jnp.int32)], ee16)

        @plsc.parallel_loop(0, K, unroll=4)
        def _row(i):
            sc = plsc.load_gather(ee_v, [jnp.full((16,), i, jnp.int32)])
            for j in range(D // 16):
                buf[i, pl.ds(j * 16, 16)] = buf[i, pl.ds(j * 16, 16)] * sc

    def _step(jj, k):
        b = k % NBUF
        b1 = (k + 1) % NBUF
        i = k % NIDX
        i1 = (k + 1) % NIDX
        i2 = (k + 2) % NIDX

        @pl.when(jj >= 2)
        def _retire():
            _wait_scatter(b1, i2)

        @pl.when(jj + 1 < NCH)
        def _pref_gather():
            _wait_idx(i1)
            _issue_gather(b1, i1)

        _wait_gather(b, i)
        _process(b)
        _issue_scatter(b, i)

        @pl.when(jj + 2 < NCH)
        def _pref_idx():
            _issue_idx(jj + 2, i2)

    _issue_idx(0, 0)
    _issue_idx(1, 1)
    _wait_idx(0)
    _issue_gather(0, 0)

    def _span(p, carry):
        for k in range(SPAN):
            jj = p * SPAN + k

            @pl.when(jj < NCH)
            def _guarded():
                _step(jj, k)

        return carry

    lax.fori_loop(0, (NCH + SPAN - 1) // SPAN, _span, 0)
    _wait_scatter((NCH - 2) % NBUF, (NCH - 2) % NIDX)
    _wait_scatter((NCH - 1) % NBUF, (NCH - 1) % NIDX)
    plsc.subcore_barrier()

    r0 = s * ROWS_PER_TILE
    pltpu.sync_copy(acc_sh.at[pl.ds(r0, ROWS_PER_TILE)],
                    out_hbm.at[pl.ds(c * NP + r0, ROWS_PER_TILE)])


def kernel(x, edge_index, W1, al1, ar1, b1, W2, al2, ar2, b2):
    src = edge_index[0]
    dst = edge_index[1]
    xp = jnp.pad(x, ((0, NP - N), (0, 0)))
    feat1, er1, sh1 = _proj(xp, W1, al1, ar1)
    acc1 = _sc_edge(feat1, er1.reshape(NP), sh1.reshape(16), src, dst)
    feat2, er2, sh2 = _norm_proj(acc1, b1.reshape(1, D), W2, al2, ar2)
    acc2 = _sc_edge(feat2, er2.reshape(NP), sh2.reshape(16), src, dst)
    out = _final(acc2, b2.reshape(1, D))
    return out[:N]


# speedup vs baseline: 1.2611x; 1.0711x over previous
import functools

import jax
import jax.numpy as jnp
from jax import lax
from jax.experimental import pallas as pl
from jax.experimental.pallas import tpu as pltpu
from jax.experimental.pallas import tpu_sc as plsc

N = 10000
NP = 10240
E = 320000
D = 128
DE = 144
NEG = 0.2
BLK = 1024
GRID = NP // BLK
NC = 2
NS = 16
NW = NC * NS
EPW = E // NW
K = 80
NCH = EPW // K
NBUF = 3
NIDX = 4
SPAN = 12
ROWS_PER_TILE = NP // NS

_F32 = jnp.float32
_HI = jax.lax.Precision.HIGHEST


def _proj_tail(fb, al_ref, ar_ref, feat_ref, er_ref, shift_ref,
               mel_ref, mer_ref):
    el = jnp.sum(fb * al_ref[...], axis=1, keepdims=True)
    er = jnp.sum(fb * ar_ref[...], axis=1, keepdims=True)
    lane = lax.broadcasted_iota(jnp.int32, (BLK, DE - D), 1)
    pad = jnp.where(lane == 0, 1.0, jnp.where(lane == 1, el, 0.0)).astype(_F32)
    feat_ref[...] = jnp.concatenate([fb, pad], axis=1)
    er_ref[...] = er.reshape(1, BLK, 1)

    i = pl.program_id(0)

    @pl.when(i == 0)
    def _init():
        mel_ref[0] = jnp.float32(-3.0e38)
        mer_ref[0] = jnp.float32(-3.0e38)

    mel_ref[0] = jnp.maximum(mel_ref[0], jnp.max(el))
    mer_ref[0] = jnp.maximum(mer_ref[0], jnp.max(er))

    @pl.when(i == GRID - 1)
    def _emit():
        m = mel_ref[0] + mer_ref[0]
        m = jnp.where(m > 0.0, m, NEG * m)
        shift_ref[...] = jnp.full((1, 1, 16), m, _F32)


def _proj_body(x_ref, w_ref, al_ref, ar_ref, feat_ref, er_ref,
               shift_ref, mel_ref, mer_ref):
    fb = jnp.dot(x_ref[...], w_ref[...], precision=_HI)
    _proj_tail(fb, al_ref, ar_ref, feat_ref, er_ref, shift_ref,
               mel_ref, mer_ref)


def _merge_norm(a0_ref, a1_ref, b_ref):
    r = a0_ref[...] + a1_ref[...]
    den = r[:, D:D + 1]
    den = jnp.where(den > 0.0, den, 1.0)
    return r[:, :D] / den + b_ref[...]


def _norm_proj_body(a0_ref, a1_ref, b_ref, w_ref, al_ref, ar_ref,
                    feat_ref, er_ref, shift_ref, mel_ref, mer_ref):
    h = _merge_norm(a0_ref, a1_ref, b_ref)
    h = jnp.where(h > 0.0, h, jnp.exp(jnp.minimum(h, 0.0)) - 1.0)
    fb = jnp.dot(h, w_ref[...], precision=_HI)
    _proj_tail(fb, al_ref, ar_ref, feat_ref, er_ref, shift_ref,
               mel_ref, mer_ref)


def _final_body(a0_ref, a1_ref, b_ref, out_ref):
    out_ref[...] = _merge_norm(a0_ref, a1_ref, b_ref)


_PROJ_OUTS = (
    jax.ShapeDtypeStruct((NP, DE), _F32),
    jax.ShapeDtypeStruct((GRID, BLK, 1), _F32),
    jax.ShapeDtypeStruct((1, 1, 16), _F32),
)
_PROJ_OUT_SPECS = [
    pl.BlockSpec((BLK, DE), lambda i: (i, 0)),
    pl.BlockSpec((1, BLK, 1), lambda i: (i, 0, 0)),
    pl.BlockSpec((1, 1, 16), lambda i: (0, 0, 0)),
]
_PROJ_SCRATCH = [pltpu.SMEM((1,), _F32), pltpu.SMEM((1,), _F32)]
_FULL2 = pl.BlockSpec((D, D), lambda i: (0, 0))
_ROW = pl.BlockSpec((1, D), lambda i: (0, 0))
_ACC0 = pl.BlockSpec((BLK, DE), lambda i: (i, 0))
_ACC1 = pl.BlockSpec((BLK, DE), lambda i: (i + GRID, 0))


def _proj(x, w, al, ar):
    return pl.pallas_call(
        _proj_body,
        grid=(GRID,),
        in_specs=[pl.BlockSpec((BLK, D), lambda i: (i, 0)), _FULL2, _ROW, _ROW],
        out_specs=_PROJ_OUT_SPECS,
        out_shape=_PROJ_OUTS,
        scratch_shapes=_PROJ_SCRATCH,
    )(x, w, al, ar)


def _norm_proj(acc, b, w, al, ar):
    return pl.pallas_call(
        _norm_proj_body,
        grid=(GRID,),
        in_specs=[_ACC0, _ACC1, _ROW, _FULL2, _ROW, _ROW],
        out_specs=_PROJ_OUT_SPECS,
        out_shape=_PROJ_OUTS,
        scratch_shapes=_PROJ_SCRATCH,
    )(acc, acc, b, w, al, ar)


def _final(acc, b):
    return pl.pallas_call(
        _final_body,
        grid=(GRID,),
        in_specs=[_ACC0, _ACC1, _ROW],
        out_specs=pl.BlockSpec((BLK, D), lambda i: (i, 0)),
        out_shape=jax.ShapeDtypeStruct((NP, D), _F32),
    )(acc, acc, b)


@functools.partial(
    pl.kernel,
    out_type=jax.ShapeDtypeStruct((NC * NP, DE), _F32),
    mesh=plsc.VectorSubcoreMesh(core_axis_name="c", subcore_axis_name="s"),
    compiler_params=pltpu.CompilerParams(
        needs_layout_passes=False, use_tc_tiling_on_sc=False),
    scratch_types=(
        [pltpu.VMEM((16,), _F32),
         pltpu.VMEM((K,), _F32)]
        + [pltpu.VMEM((K,), jnp.int32),
           pltpu.VMEM((K,), jnp.int32),
           pltpu.SemaphoreType.DMA] * NIDX
        + [pltpu.VMEM((K, DE), _F32),
           pltpu.VMEM((K,), _F32),
           pltpu.SemaphoreType.DMA,
           pltpu.SemaphoreType.DMA] * NBUF
        + [pltpu.VMEM_SHARED((NP, DE), _F32)]
    ),
)
def _sc_edge(feat_hbm, er_hbm, shift_hbm, src_hbm, dst_hbm, out_hbm,
             shift_v, ee_v, *rest):
    idxs = [rest[3 * i:3 * i + 3] for i in range(NIDX)]
    rows = [rest[3 * NIDX + 4 * b:3 * NIDX + 4 * b + 4] for b in range(NBUF)]
    acc_sh = rest[3 * NIDX + 4 * NBUF]
    c = lax.axis_index("c")
    s = lax.axis_index("s")
    wid = c * NS + s
    buf0 = rows[0][0]

    @plsc.parallel_loop(0, K, unroll=4)
    def _zrow(i):
        for j in range(DE // 16):
            buf0[i, pl.ds(j * 16, 16)] = jnp.zeros((16,), _F32)
    for t in range(ROWS_PER_TILE // K):
        pltpu.sync_copy(buf0, acc_sh.at[pl.ds(s * ROWS_PER_TILE + t * K, K)])

    pltpu.sync_copy(shift_hbm, shift_v)
    plsc.subcore_barrier()
    shift = shift_v[...]

    def _issue_idx(jj, i):
        base = wid * EPW + jj * K
        src_v, dst_v, isem = idxs[i]
        pltpu.async_copy(src_hbm.at[pl.ds(base, K)], src_v, isem)
        pltpu.async_copy(dst_hbm.at[pl.ds(base, K)], dst_v, isem)

    def _wait_idx(i):
        src_v, dst_v, isem = idxs[i]
        pltpu.make_async_copy(src_hbm.at[pl.ds(0, K)], src_v, isem).wait()
        pltpu.make_async_copy(dst_hbm.at[pl.ds(0, K)], dst_v, isem).wait()

    def _issue_gather(b, i):
        buf, erg, gsem, _ = rows[b]
        pltpu.async_copy(feat_hbm.at[idxs[i][0]], buf, gsem)
        pltpu.async_copy(er_hbm.at[idxs[i][1]], erg, gsem)

    def _wait_gather(b, i):
        buf, erg, gsem, _ = rows[b]
        pltpu.make_async_copy(feat_hbm.at[idxs[i][0]], buf, gsem).wait()
        pltpu.make_async_copy(er_hbm.at[idxs[i][1]], erg, gsem).wait()

    def _issue_scatter(b, i):
        buf, _, _, ssem = rows[b]
        pltpu.async_copy(buf, acc_sh.at[idxs[i][1]], ssem, add=True)

    def _wait_scatter(b, i):
        buf, _, _, ssem = rows[b]
        pltpu.make_async_copy(buf, acc_sh.at[idxs[i][1]], ssem).wait()

    def _process(b):
        buf, erg, _, _ = rows[b]

        @plsc.parallel_loop(0, K // 16, unroll=2)
        def _ee(i):
            grp = lax.iota(jnp.int32, 16) + i * 16
            elg = plsc.load_gather(buf, [grp, jnp.full((16,), D + 1, jnp.int32)])
            e = elg + erg[pl.ds(i * 16, 16)]
            e = jnp.where(e > 0.0, e, NEG * e) - shift
            ee16 = jnp.exp(e)
            ee_v[pl.ds(i * 16, 16)] = ee16
            plsc.store_scatter(buf, [grp, jnp.full((16,), D, jnp.int32)], ee16)

        @plsc.parallel_loop(0, K, unroll=4)
        def _row(i):
            sc = plsc.load_gather(ee_v, [jnp.full((16,), i, jnp.int32)])
            for j in range(D // 16):
                buf[i, pl.ds(j * 16, 16)] = buf[i, pl.ds(j * 16, 16)] * sc

    def _step(jj, k):
        b = k % NBUF
        b1 = (k + 1) % NBUF
        i = k % NIDX
        i1 = (k + 1) % NIDX
        i2 = (k + 2) % NIDX

        @pl.when(jj >= 2)
        def _retire():
            _wait_scatter(b1, i2)

        @pl.when(jj + 1 < NCH)
        def _pref_gather():
            _wait_idx(i1)
            _issue_gather(b1, i1)

        _wait_gather(b, i)
        _process(b)
        _issue_scatter(b, i)

        @pl.when(jj + 2 < NCH)
        def _pref_idx():
            _issue_idx(jj + 2, i2)

    _issue_idx(0, 0)
    _issue_idx(1, 1)
    _wait_idx(0)
    _issue_gather(0, 0)

    def _span(p, carry):
        for k in range(SPAN):
            jj = p * SPAN + k

            @pl.when(jj < NCH)
            def _guarded():
                _step(jj, k)

        return carry

    lax.fori_loop(0, (NCH + SPAN - 1) // SPAN, _span, 0)
    _wait_scatter((NCH - 2) % NBUF, (NCH - 2) % NIDX)
    _wait_scatter((NCH - 1) % NBUF, (NCH - 1) % NIDX)
    plsc.subcore_barrier()

    r0 = s * ROWS_PER_TILE
    pltpu.sync_copy(acc_sh.at[pl.ds(r0, ROWS_PER_TILE)],
                    out_hbm.at[pl.ds(c * NP + r0, ROWS_PER_TILE)])


def kernel(x, edge_index, W1, al1, ar1, b1, W2, al2, ar2, b2):
    src = edge_index[0]
    dst = edge_index[1]
    xp = jnp.pad(x, ((0, NP - N), (0, 0)))
    feat1, er1, sh1 = _proj(xp, W1, al1, ar1)
    acc1 = _sc_edge(feat1, er1.reshape(NP), sh1.reshape(16), src, dst)
    feat2, er2, sh2 = _norm_proj(acc1, b1.reshape(1, D), W2, al2, ar2)
    acc2 = _sc_edge(feat2, er2.reshape(NP), sh2.reshape(16), src, dst)
    out = _final(acc2, b2.reshape(1, D))
    return out[:N]


# speedup vs baseline: 1.2773x; 1.0128x over previous
import functools

import jax
import jax.numpy as jnp
from jax import lax
from jax.experimental import pallas as pl
from jax.experimental.pallas import tpu as pltpu
from jax.experimental.pallas import tpu_sc as plsc

N = 10000
NP = 10240
E = 320000
D = 128
DE = 144
NEG = 0.2
BLK = 2048
GRID = NP // BLK
NC = 2
NS = 16
NW = NC * NS
EPW = E // NW
K = 80
NCH = EPW // K
NBUF = 3
NIDX = 4
SPAN = 12
ROWS_PER_TILE = NP // NS

_F32 = jnp.float32
_HI = jax.lax.Precision.HIGHEST


def _proj_tail(fb, al_ref, ar_ref, feat_ref, er_ref, shift_ref,
               mel_ref, mer_ref):
    el = jnp.sum(fb * al_ref[...], axis=1, keepdims=True)
    er = jnp.sum(fb * ar_ref[...], axis=1, keepdims=True)
    lane = lax.broadcasted_iota(jnp.int32, (BLK, DE - D), 1)
    pad = jnp.where(lane == 0, 1.0, jnp.where(lane == 1, el, 0.0)).astype(_F32)
    feat_ref[...] = jnp.concatenate([fb, pad], axis=1)
    er_ref[...] = er.reshape(1, BLK, 1)

    i = pl.program_id(0)

    @pl.when(i == 0)
    def _init():
        mel_ref[0] = jnp.float32(-3.0e38)
        mer_ref[0] = jnp.float32(-3.0e38)

    mel_ref[0] = jnp.maximum(mel_ref[0], jnp.max(el))
    mer_ref[0] = jnp.maximum(mer_ref[0], jnp.max(er))

    @pl.when(i == GRID - 1)
    def _emit():
        m = mel_ref[0] + mer_ref[0]
        m = jnp.where(m > 0.0, m, NEG * m)
        shift_ref[...] = jnp.full((1, 1, 16), m, _F32)


def _proj_body(x_ref, w_ref, al_ref, ar_ref, feat_ref, er_ref,
               shift_ref, mel_ref, mer_ref):
    fb = jnp.dot(x_ref[...], w_ref[...], precision=_HI)
    _proj_tail(fb, al_ref, ar_ref, feat_ref, er_ref, shift_ref,
               mel_ref, mer_ref)


def _merge_norm(a0_ref, a1_ref, b_ref):
    r = a0_ref[...] + a1_ref[...]
    den = r[:, D:D + 1]
    den = jnp.where(den > 0.0, den, 1.0)
    return r[:, :D] / den + b_ref[...]


def _norm_proj_body(a0_ref, a1_ref, b_ref, w_ref, al_ref, ar_ref,
                    feat_ref, er_ref, shift_ref, mel_ref, mer_ref):
    h = _merge_norm(a0_ref, a1_ref, b_ref)
    h = jnp.where(h > 0.0, h, jnp.exp(jnp.minimum(h, 0.0)) - 1.0)
    fb = jnp.dot(h, w_ref[...], precision=_HI)
    _proj_tail(fb, al_ref, ar_ref, feat_ref, er_ref, shift_ref,
               mel_ref, mer_ref)


def _final_body(a0_ref, a1_ref, b_ref, out_ref):
    out_ref[...] = _merge_norm(a0_ref, a1_ref, b_ref)


_PROJ_OUTS = (
    jax.ShapeDtypeStruct((NP, DE), _F32),
    jax.ShapeDtypeStruct((GRID, BLK, 1), _F32),
    jax.ShapeDtypeStruct((1, 1, 16), _F32),
)
_PROJ_OUT_SPECS = [
    pl.BlockSpec((BLK, DE), lambda i: (i, 0)),
    pl.BlockSpec((1, BLK, 1), lambda i: (i, 0, 0)),
    pl.BlockSpec((1, 1, 16), lambda i: (0, 0, 0)),
]
_PROJ_SCRATCH = [pltpu.SMEM((1,), _F32), pltpu.SMEM((1,), _F32)]
_FULL2 = pl.BlockSpec((D, D), lambda i: (0, 0))
_ROW = pl.BlockSpec((1, D), lambda i: (0, 0))
_ACC0 = pl.BlockSpec((BLK, DE), lambda i: (i, 0))
_ACC1 = pl.BlockSpec((BLK, DE), lambda i: (i + GRID, 0))


def _proj(x, w, al, ar):
    return pl.pallas_call(
        _proj_body,
        grid=(GRID,),
        in_specs=[pl.BlockSpec((BLK, D), lambda i: (i, 0)), _FULL2, _ROW, _ROW],
        out_specs=_PROJ_OUT_SPECS,
        out_shape=_PROJ_OUTS,
        scratch_shapes=_PROJ_SCRATCH,
    )(x, w, al, ar)


def _norm_proj(acc, b, w, al, ar):
    return pl.pallas_call(
        _norm_proj_body,
        grid=(GRID,),
        in_specs=[_ACC0, _ACC1, _ROW, _FULL2, _ROW, _ROW],
        out_specs=_PROJ_OUT_SPECS,
        out_shape=_PROJ_OUTS,
        scratch_shapes=_PROJ_SCRATCH,
    )(acc, acc, b, w, al, ar)


def _final(acc, b):
    return pl.pallas_call(
        _final_body,
        grid=(GRID,),
        in_specs=[_ACC0, _ACC1, _ROW],
        out_specs=pl.BlockSpec((BLK, D), lambda i: (i, 0)),
        out_shape=jax.ShapeDtypeStruct((NP, D), _F32),
    )(acc, acc, b)


@functools.partial(
    pl.kernel,
    out_type=jax.ShapeDtypeStruct((NC * NP, DE), _F32),
    mesh=plsc.VectorSubcoreMesh(core_axis_name="c", subcore_axis_name="s"),
    compiler_params=pltpu.CompilerParams(
        needs_layout_passes=False, use_tc_tiling_on_sc=False),
    scratch_types=(
        [pltpu.VMEM((16,), _F32),
         pltpu.VMEM((K,), _F32)]
        + [pltpu.VMEM((K,), jnp.int32),
           pltpu.VMEM((K,), jnp.int32),
           pltpu.SemaphoreType.DMA] * NIDX
        + [pltpu.VMEM((K, DE), _F32),
           pltpu.VMEM((K,), _F32),
           pltpu.SemaphoreType.DMA,
           pltpu.SemaphoreType.DMA] * NBUF
        + [pltpu.VMEM_SHARED((NP, DE), _F32)]
    ),
)
def _sc_edge(feat_hbm, er_hbm, shift_hbm, src_hbm, dst_hbm, out_hbm,
             shift_v, ee_v, *rest):
    idxs = [rest[3 * i:3 * i + 3] for i in range(NIDX)]
    rows = [rest[3 * NIDX + 4 * b:3 * NIDX + 4 * b + 4] for b in range(NBUF)]
    acc_sh = rest[3 * NIDX + 4 * NBUF]
    c = lax.axis_index("c")
    s = lax.axis_index("s")
    wid = c * NS + s
    buf0 = rows[0][0]

    @plsc.parallel_loop(0, K, unroll=4)
    def _zrow(i):
        for j in range(DE // 16):
            buf0[i, pl.ds(j * 16, 16)] = jnp.zeros((16,), _F32)
    for t in range(ROWS_PER_TILE // K):
        pltpu.sync_copy(buf0, acc_sh.at[pl.ds(s * ROWS_PER_TILE + t * K, K)])

    pltpu.sync_copy(shift_hbm, shift_v)
    plsc.subcore_barrier()
    shift = shift_v[...]

    def _issue_idx(jj, i):
        base = wid * EPW + jj * K
        src_v, dst_v, isem = idxs[i]
        pltpu.async_copy(src_hbm.at[pl.ds(base, K)], src_v, isem)
        pltpu.async_copy(dst_hbm.at[pl.ds(base, K)], dst_v, isem)

    def _wait_idx(i):
        src_v, dst_v, isem = idxs[i]
        pltpu.make_async_copy(src_hbm.at[pl.ds(0, K)], src_v, isem).wait()
        pltpu.make_async_copy(dst_hbm.at[pl.ds(0, K)], dst_v, isem).wait()

    def _issue_gather(b, i):
        buf, erg, gsem, _ = rows[b]
        pltpu.async_copy(feat_hbm.at[idxs[i][0]], buf, gsem)
        pltpu.async_copy(er_hbm.at[idxs[i][1]], erg, gsem)

    def _wait_gather(b, i):
        buf, erg, gsem, _ = rows[b]
        pltpu.make_async_copy(feat_hbm.at[idxs[i][0]], buf, gsem).wait()
        pltpu.make_async_copy(er_hbm.at[idxs[i][1]], erg, gsem).wait()

    def _issue_scatter(b, i):
        buf, _, _, ssem = rows[b]
        pltpu.async_copy(buf, acc_sh.at[idxs[i][1]], ssem, add=True)

    def _wait_scatter(b, i):
        buf, _, _, ssem = rows[b]
        pltpu.make_async_copy(buf, acc_sh.at[idxs[i][1]], ssem).wait()

    def _process(b):
        buf, erg, _, _ = rows[b]

        @plsc.parallel_loop(0, K // 16, unroll=2)
        def _ee(i):
            grp = lax.iota(jnp.int32, 16) + i * 16
            elg = plsc.load_gather(buf, [grp, jnp.full((16,), D + 1, jnp.int32)])
            e = elg + erg[pl.ds(i * 16, 16)]
            e = jnp.where(e > 0.0, e, NEG * e) - shift
            ee16 = jnp.exp(e)
            ee_v[pl.ds(i * 16, 16)] = ee16
            plsc.store_scatter(buf, [grp, jnp.full((16,), D, jnp.int32)], ee16)

        @plsc.parallel_loop(0, K, unroll=4)
        def _row(i):
            sc = plsc.load_gather(ee_v, [jnp.full((16,), i, jnp.int32)])
            for j in range(D // 16):
                buf[i, pl.ds(j * 16, 16)] = buf[i, pl.ds(j * 16, 16)] * sc

    def _step(jj, k):
        b = k % NBUF
        b1 = (k + 1) % NBUF
        i = k % NIDX
        i1 = (k + 1) % NIDX
        i2 = (k + 2) % NIDX

        @pl.when(jj >= 2)
        def _retire():
            _wait_scatter(b1, i2)

        @pl.when(jj + 1 < NCH)
        def _pref_gather():
            _wait_idx(i1)
            _issue_gather(b1, i1)

        _wait_gather(b, i)
        _process(b)
        _issue_scatter(b, i)

        @pl.when(jj + 2 < NCH)
        def _pref_idx():
            _issue_idx(jj + 2, i2)

    _issue_idx(0, 0)
    _issue_idx(1, 1)
    _wait_idx(0)
    _issue_gather(0, 0)

    def _span(p, carry):
        for k in range(SPAN):
            jj = p * SPAN + k

            @pl.when(jj < NCH)
            def _guarded():
                _step(jj, k)

        return carry

    lax.fori_loop(0, (NCH + SPAN - 1) // SPAN, _span, 0)
    _wait_scatter((NCH - 2) % NBUF, (NCH - 2) % NIDX)
    _wait_scatter((NCH - 1) % NBUF, (NCH - 1) % NIDX)
    plsc.subcore_barrier()

    r0 = s * ROWS_PER_TILE
    pltpu.sync_copy(acc_sh.at[pl.ds(r0, ROWS_PER_TILE)],
                    out_hbm.at[pl.ds(c * NP + r0, ROWS_PER_TILE)])


def kernel(x, edge_index, W1, al1, ar1, b1, W2, al2, ar2, b2):
    src = edge_index[0]
    dst = edge_index[1]
    xp = jnp.pad(x, ((0, NP - N), (0, 0)))
    feat1, er1, sh1 = _proj(xp, W1, al1, ar1)
    acc1 = _sc_edge(feat1, er1.reshape(NP), sh1.reshape(16), src, dst)
    feat2, er2, sh2 = _norm_proj(acc1, b1.reshape(1, D), W2, al2, ar2)
    acc2 = _sc_edge(feat2, er2.reshape(NP), sh2.reshape(16), src, dst)
    out = _final(acc2, b2.reshape(1, D))
    return out[:N]


# speedup vs baseline: 1.2820x; 1.0037x over previous
import functools

import jax
import jax.numpy as jnp
from jax import lax
from jax.experimental import pallas as pl
from jax.experimental.pallas import tpu as pltpu
from jax.experimental.pallas import tpu_sc as plsc

N = 10000
NP = 10240
E = 320000
D = 128
DE = 144
NEG = 0.2
BLK = 2560
GRID = NP // BLK
NC = 2
NS = 16
NW = NC * NS
EPW = E // NW
K = 80
NCH = EPW // K
NBUF = 3
NIDX = 4
SPAN = 12
ROWS_PER_TILE = NP // NS

_F32 = jnp.float32
_HI = jax.lax.Precision.HIGHEST


def _proj_tail(fb, al_ref, ar_ref, feat_ref, er_ref, shift_ref,
               mel_ref, mer_ref):
    el = jnp.sum(fb * al_ref[...], axis=1, keepdims=True)
    er = jnp.sum(fb * ar_ref[...], axis=1, keepdims=True)
    lane = lax.broadcasted_iota(jnp.int32, (BLK, DE - D), 1)
    pad = jnp.where(lane == 0, 1.0, jnp.where(lane == 1, el, 0.0)).astype(_F32)
    feat_ref[...] = jnp.concatenate([fb, pad], axis=1)
    er_ref[...] = er.reshape(1, BLK, 1)

    i = pl.program_id(0)

    @pl.when(i == 0)
    def _init():
        mel_ref[0] = jnp.float32(-3.0e38)
        mer_ref[0] = jnp.float32(-3.0e38)

    mel_ref[0] = jnp.maximum(mel_ref[0], jnp.max(el))
    mer_ref[0] = jnp.maximum(mer_ref[0], jnp.max(er))

    @pl.when(i == GRID - 1)
    def _emit():
        m = mel_ref[0] + mer_ref[0]
        m = jnp.where(m > 0.0, m, NEG * m)
        shift_ref[...] = jnp.full((1, 1, 16), m, _F32)


def _proj_body(x_ref, w_ref, al_ref, ar_ref, feat_ref, er_ref,
               shift_ref, mel_ref, mer_ref):
    fb = jnp.dot(x_ref[...], w_ref[...], precision=_HI)
    _proj_tail(fb, al_ref, ar_ref, feat_ref, er_ref, shift_ref,
               mel_ref, mer_ref)


def _merge_norm(a0_ref, a1_ref, b_ref):
    r = a0_ref[...] + a1_ref[...]
    den = r[:, D:D + 1]
    den = jnp.where(den > 0.0, den, 1.0)
    return r[:, :D] / den + b_ref[...]


def _norm_proj_body(a0_ref, a1_ref, b_ref, w_ref, al_ref, ar_ref,
                    feat_ref, er_ref, shift_ref, mel_ref, mer_ref):
    h = _merge_norm(a0_ref, a1_ref, b_ref)
    h = jnp.where(h > 0.0, h, jnp.exp(jnp.minimum(h, 0.0)) - 1.0)
    fb = jnp.dot(h, w_ref[...], precision=_HI)
    _proj_tail(fb, al_ref, ar_ref, feat_ref, er_ref, shift_ref,
               mel_ref, mer_ref)


def _final_body(a0_ref, a1_ref, b_ref, out_ref):
    out_ref[...] = _merge_norm(a0_ref, a1_ref, b_ref)


_PROJ_OUTS = (
    jax.ShapeDtypeStruct((NP, DE), _F32),
    jax.ShapeDtypeStruct((GRID, BLK, 1), _F32),
    jax.ShapeDtypeStruct((1, 1, 16), _F32),
)
_PROJ_OUT_SPECS = [
    pl.BlockSpec((BLK, DE), lambda i: (i, 0)),
    pl.BlockSpec((1, BLK, 1), lambda i: (i, 0, 0)),
    pl.BlockSpec((1, 1, 16), lambda i: (0, 0, 0)),
]
_PROJ_SCRATCH = [pltpu.SMEM((1,), _F32), pltpu.SMEM((1,), _F32)]
_FULL2 = pl.BlockSpec((D, D), lambda i: (0, 0))
_ROW = pl.BlockSpec((1, D), lambda i: (0, 0))
_ACC0 = pl.BlockSpec((BLK, DE), lambda i: (i, 0))
_ACC1 = pl.BlockSpec((BLK, DE), lambda i: (i + GRID, 0))


def _proj(x, w, al, ar):
    return pl.pallas_call(
        _proj_body,
        grid=(GRID,),
        in_specs=[pl.BlockSpec((BLK, D), lambda i: (i, 0)), _FULL2, _ROW, _ROW],
        out_specs=_PROJ_OUT_SPECS,
        out_shape=_PROJ_OUTS,
        scratch_shapes=_PROJ_SCRATCH,
    )(x, w, al, ar)


def _norm_proj(acc, b, w, al, ar):
    return pl.pallas_call(
        _norm_proj_body,
        grid=(GRID,),
        in_specs=[_ACC0, _ACC1, _ROW, _FULL2, _ROW, _ROW],
        out_specs=_PROJ_OUT_SPECS,
        out_shape=_PROJ_OUTS,
        scratch_shapes=_PROJ_SCRATCH,
    )(acc, acc, b, w, al, ar)


def _final(acc, b):
    return pl.pallas_call(
        _final_body,
        grid=(GRID,),
        in_specs=[_ACC0, _ACC1, _ROW],
        out_specs=pl.BlockSpec((BLK, D), lambda i: (i, 0)),
        out_shape=jax.ShapeDtypeStruct((NP, D), _F32),
    )(acc, acc, b)


@functools.partial(
    pl.kernel,
    out_type=jax.ShapeDtypeStruct((NC * NP, DE), _F32),
    mesh=plsc.VectorSubcoreMesh(core_axis_name="c", subcore_axis_name="s"),
    compiler_params=pltpu.CompilerParams(
        needs_layout_passes=False, use_tc_tiling_on_sc=False),
    scratch_types=(
        [pltpu.VMEM((16,), _F32),
         pltpu.VMEM((K,), _F32)]
        + [pltpu.VMEM((K,), jnp.int32),
           pltpu.VMEM((K,), jnp.int32),
           pltpu.SemaphoreType.DMA] * NIDX
        + [pltpu.VMEM((K, DE), _F32),
           pltpu.VMEM((K,), _F32),
           pltpu.SemaphoreType.DMA,
           pltpu.SemaphoreType.DMA] * NBUF
        + [pltpu.VMEM_SHARED((NP, DE), _F32)]
    ),
)
def _sc_edge(feat_hbm, er_hbm, shift_hbm, src_hbm, dst_hbm, out_hbm,
             shift_v, ee_v, *rest):
    idxs = [rest[3 * i:3 * i + 3] for i in range(NIDX)]
    rows = [rest[3 * NIDX + 4 * b:3 * NIDX + 4 * b + 4] for b in range(NBUF)]
    acc_sh = rest[3 * NIDX + 4 * NBUF]
    c = lax.axis_index("c")
    s = lax.axis_index("s")
    wid = c * NS + s
    buf0 = rows[0][0]

    @plsc.parallel_loop(0, K, unroll=4)
    def _zrow(i):
        for j in range(DE // 16):
            buf0[i, pl.ds(j * 16, 16)] = jnp.zeros((16,), _F32)
    for t in range(ROWS_PER_TILE // K):
        pltpu.sync_copy(buf0, acc_sh.at[pl.ds(s * ROWS_PER_TILE + t * K, K)])

    pltpu.sync_copy(shift_hbm, shift_v)
    plsc.subcore_barrier()
    shift = shift_v[...]

    def _issue_idx(jj, i):
        base = wid * EPW + jj * K
        src_v, dst_v, isem = idxs[i]
        pltpu.async_copy(src_hbm.at[pl.ds(base, K)], src_v, isem)
        pltpu.async_copy(dst_hbm.at[pl.ds(base, K)], dst_v, isem)

    def _wait_idx(i):
        src_v, dst_v, isem = idxs[i]
        pltpu.make_async_copy(src_hbm.at[pl.ds(0, K)], src_v, isem).wait()
        pltpu.make_async_copy(dst_hbm.at[pl.ds(0, K)], dst_v, isem).wait()

    def _issue_gather(b, i):
        buf, erg, gsem, _ = rows[b]
        pltpu.async_copy(feat_hbm.at[idxs[i][0]], buf, gsem)
        pltpu.async_copy(er_hbm.at[idxs[i][1]], erg, gsem)

    def _wait_gather(b, i):
        buf, erg, gsem, _ = rows[b]
        pltpu.make_async_copy(feat_hbm.at[idxs[i][0]], buf, gsem).wait()
        pltpu.make_async_copy(er_hbm.at[idxs[i][1]], erg, gsem).wait()

    def _issue_scatter(b, i):
        buf, _, _, ssem = rows[b]
        pltpu.async_copy(buf, acc_sh.at[idxs[i][1]], ssem, add=True)

    def _wait_scatter(b, i):
        buf, _, _, ssem = rows[b]
        pltpu.make_async_copy(buf, acc_sh.at[idxs[i][1]], ssem).wait()

    def _process(b):
        buf, erg, _, _ = rows[b]

        @plsc.parallel_loop(0, K // 16, unroll=2)
        def _ee(i):
            grp = lax.iota(jnp.int32, 16) + i * 16
            elg = plsc.load_gather(buf, [grp, jnp.full((16,), D + 1, jnp.int32)])
            e = elg + erg[pl.ds(i * 16, 16)]
            e = jnp.where(e > 0.0, e, NEG * e) - shift
            ee16 = jnp.exp(e)
            ee_v[pl.ds(i * 16, 16)] = ee16
            plsc.store_scatter(buf, [grp, jnp.full((16,), D, jnp.int32)], ee16)

        @plsc.parallel_loop(0, K, unroll=4)
        def _row(i):
            sc = plsc.load_gather(ee_v, [jnp.full((16,), i, jnp.int32)])
            for j in range(D // 16):
                buf[i, pl.ds(j * 16, 16)] = buf[i, pl.ds(j * 16, 16)] * sc

    def _step(jj, k):
        b = k % NBUF
        b1 = (k + 1) % NBUF
        i = k % NIDX
        i1 = (k + 1) % NIDX
        i2 = (k + 2) % NIDX

        @pl.when(jj >= 2)
        def _retire():
            _wait_scatter(b1, i2)

        @pl.when(jj + 1 < NCH)
        def _pref_gather():
            _wait_idx(i1)
            _issue_gather(b1, i1)

        _wait_gather(b, i)
        _process(b)
        _issue_scatter(b, i)

        @pl.when(jj + 2 < NCH)
        def _pref_idx():
            _issue_idx(jj + 2, i2)

    _issue_idx(0, 0)
    _issue_idx(1, 1)
    _wait_idx(0)
    _issue_gather(0, 0)

    def _span(p, carry):
        for k in range(SPAN):
            jj = p * SPAN + k

            @pl.when(jj < NCH)
            def _guarded():
                _step(jj, k)

        return carry

    lax.fori_loop(0, (NCH + SPAN - 1) // SPAN, _span, 0)
    _wait_scatter((NCH - 2) % NBUF, (NCH - 2) % NIDX)
    _wait_scatter((NCH - 1) % NBUF, (NCH - 1) % NIDX)
    plsc.subcore_barrier()

    r0 = s * ROWS_PER_TILE
    pltpu.sync_copy(acc_sh.at[pl.ds(r0, ROWS_PER_TILE)],
                    out_hbm.at[pl.ds(c * NP + r0, ROWS_PER_TILE)])


def kernel(x, edge_index, W1, al1, ar1, b1, W2, al2, ar2, b2):
    src = edge_index[0]
    dst = edge_index[1]
    xp = jnp.pad(x, ((0, NP - N), (0, 0)))
    feat1, er1, sh1 = _proj(xp, W1, al1, ar1)
    acc1 = _sc_edge(feat1, er1.reshape(NP), sh1.reshape(16), src, dst)
    feat2, er2, sh2 = _norm_proj(acc1, b1.reshape(1, D), W2, al2, ar2)
    acc2 = _sc_edge(feat2, er2.reshape(NP), sh2.reshape(16), src, dst)
    out = _final(acc2, b2.reshape(1, D))
    return out[:N]


# speedup vs baseline: 1.2836x; 1.0012x over previous
import functools

import jax
import jax.numpy as jnp
from jax import lax
from jax.experimental import pallas as pl
from jax.experimental.pallas import tpu as pltpu
from jax.experimental.pallas import tpu_sc as plsc

N = 10000
NP = 10240
E = 320000
D = 128
DE = 144
NEG = 0.2
BLK = 5120
GRID = NP // BLK
NC = 2
NS = 16
NW = NC * NS
EPW = E // NW
K = 80
NCH = EPW // K
NBUF = 3
NIDX = 4
SPAN = 12
ROWS_PER_TILE = NP // NS

_F32 = jnp.float32
_HI = jax.lax.Precision.HIGHEST


def _proj_tail(fb, al_ref, ar_ref, feat_ref, er_ref, shift_ref,
               mel_ref, mer_ref):
    el = jnp.sum(fb * al_ref[...], axis=1, keepdims=True)
    er = jnp.sum(fb * ar_ref[...], axis=1, keepdims=True)
    lane = lax.broadcasted_iota(jnp.int32, (BLK, DE - D), 1)
    pad = jnp.where(lane == 0, 1.0, jnp.where(lane == 1, el, 0.0)).astype(_F32)
    feat_ref[...] = jnp.concatenate([fb, pad], axis=1)
    er_ref[...] = er.reshape(1, BLK, 1)

    i = pl.program_id(0)

    @pl.when(i == 0)
    def _init():
        mel_ref[0] = jnp.float32(-3.0e38)
        mer_ref[0] = jnp.float32(-3.0e38)

    mel_ref[0] = jnp.maximum(mel_ref[0], jnp.max(el))
    mer_ref[0] = jnp.maximum(mer_ref[0], jnp.max(er))

    @pl.when(i == GRID - 1)
    def _emit():
        m = mel_ref[0] + mer_ref[0]
        m = jnp.where(m > 0.0, m, NEG * m)
        shift_ref[...] = jnp.full((1, 1, 16), m, _F32)


def _proj_body(x_ref, w_ref, al_ref, ar_ref, feat_ref, er_ref,
               shift_ref, mel_ref, mer_ref):
    fb = jnp.dot(x_ref[...], w_ref[...], precision=_HI)
    _proj_tail(fb, al_ref, ar_ref, feat_ref, er_ref, shift_ref,
               mel_ref, mer_ref)


def _merge_norm(a0_ref, a1_ref, b_ref):
    r = a0_ref[...] + a1_ref[...]
    den = r[:, D:D + 1]
    den = jnp.where(den > 0.0, den, 1.0)
    return r[:, :D] / den + b_ref[...]


def _norm_proj_body(a0_ref, a1_ref, b_ref, w_ref, al_ref, ar_ref,
                    feat_ref, er_ref, shift_ref, mel_ref, mer_ref):
    h = _merge_norm(a0_ref, a1_ref, b_ref)
    h = jnp.where(h > 0.0, h, jnp.exp(jnp.minimum(h, 0.0)) - 1.0)
    fb = jnp.dot(h, w_ref[...], precision=_HI)
    _proj_tail(fb, al_ref, ar_ref, feat_ref, er_ref, shift_ref,
               mel_ref, mer_ref)


def _final_body(a0_ref, a1_ref, b_ref, out_ref):
    out_ref[...] = _merge_norm(a0_ref, a1_ref, b_ref)


_PROJ_OUTS = (
    jax.ShapeDtypeStruct((NP, DE), _F32),
    jax.ShapeDtypeStruct((GRID, BLK, 1), _F32),
    jax.ShapeDtypeStruct((1, 1, 16), _F32),
)
_PROJ_OUT_SPECS = [
    pl.BlockSpec((BLK, DE), lambda i: (i, 0)),
    pl.BlockSpec((1, BLK, 1), lambda i: (i, 0, 0)),
    pl.BlockSpec((1, 1, 16), lambda i: (0, 0, 0)),
]
_PROJ_SCRATCH = [pltpu.SMEM((1,), _F32), pltpu.SMEM((1,), _F32)]
_FULL2 = pl.BlockSpec((D, D), lambda i: (0, 0))
_ROW = pl.BlockSpec((1, D), lambda i: (0, 0))
_ACC0 = pl.BlockSpec((BLK, DE), lambda i: (i, 0))
_ACC1 = pl.BlockSpec((BLK, DE), lambda i: (i + GRID, 0))


def _proj(x, w, al, ar):
    return pl.pallas_call(
        _proj_body,
        grid=(GRID,),
        in_specs=[pl.BlockSpec((BLK, D), lambda i: (i, 0)), _FULL2, _ROW, _ROW],
        out_specs=_PROJ_OUT_SPECS,
        out_shape=_PROJ_OUTS,
        scratch_shapes=_PROJ_SCRATCH,
    )(x, w, al, ar)


def _norm_proj(acc, b, w, al, ar):
    return pl.pallas_call(
        _norm_proj_body,
        grid=(GRID,),
        in_specs=[_ACC0, _ACC1, _ROW, _FULL2, _ROW, _ROW],
        out_specs=_PROJ_OUT_SPECS,
        out_shape=_PROJ_OUTS,
        scratch_shapes=_PROJ_SCRATCH,
    )(acc, acc, b, w, al, ar)


def _final(acc, b):
    return pl.pallas_call(
        _final_body,
        grid=(GRID,),
        in_specs=[_ACC0, _ACC1, _ROW],
        out_specs=pl.BlockSpec((BLK, D), lambda i: (i, 0)),
        out_shape=jax.ShapeDtypeStruct((NP, D), _F32),
    )(acc, acc, b)


@functools.partial(
    pl.kernel,
    out_type=jax.ShapeDtypeStruct((NC * NP, DE), _F32),
    mesh=plsc.VectorSubcoreMesh(core_axis_name="c", subcore_axis_name="s"),
    compiler_params=pltpu.CompilerParams(
        needs_layout_passes=False, use_tc_tiling_on_sc=False),
    scratch_types=(
        [pltpu.VMEM((16,), _F32),
         pltpu.VMEM((K,), _F32)]
        + [pltpu.VMEM((K,), jnp.int32),
           pltpu.VMEM((K,), jnp.int32),
           pltpu.SemaphoreType.DMA] * NIDX
        + [pltpu.VMEM((K, DE), _F32),
           pltpu.VMEM((K,), _F32),
           pltpu.SemaphoreType.DMA,
           pltpu.SemaphoreType.DMA] * NBUF
        + [pltpu.VMEM_SHARED((NP, DE), _F32)]
    ),
)
def _sc_edge(feat_hbm, er_hbm, shift_hbm, src_hbm, dst_hbm, out_hbm,
             shift_v, ee_v, *rest):
    idxs = [rest[3 * i:3 * i + 3] for i in range(NIDX)]
    rows = [rest[3 * NIDX + 4 * b:3 * NIDX + 4 * b + 4] for b in range(NBUF)]
    acc_sh = rest[3 * NIDX + 4 * NBUF]
    c = lax.axis_index("c")
    s = lax.axis_index("s")
    wid = c * NS + s
    buf0 = rows[0][0]

    @plsc.parallel_loop(0, K, unroll=4)
    def _zrow(i):
        for j in range(DE // 16):
            buf0[i, pl.ds(j * 16, 16)] = jnp.zeros((16,), _F32)
    for t in range(ROWS_PER_TILE // K):
        pltpu.sync_copy(buf0, acc_sh.at[pl.ds(s * ROWS_PER_TILE + t * K, K)])

    pltpu.sync_copy(shift_hbm, shift_v)
    plsc.subcore_barrier()
    shift = shift_v[...]

    def _issue_idx(jj, i):
        base = wid * EPW + jj * K
        src_v, dst_v, isem = idxs[i]
        pltpu.async_copy(src_hbm.at[pl.ds(base, K)], src_v, isem)
        pltpu.async_copy(dst_hbm.at[pl.ds(base, K)], dst_v, isem)

    def _wait_idx(i):
        src_v, dst_v, isem = idxs[i]
        pltpu.make_async_copy(src_hbm.at[pl.ds(0, K)], src_v, isem).wait()
        pltpu.make_async_copy(dst_hbm.at[pl.ds(0, K)], dst_v, isem).wait()

    def _issue_gather(b, i):
        buf, erg, gsem, _ = rows[b]
        pltpu.async_copy(feat_hbm.at[idxs[i][0]], buf, gsem)
        pltpu.async_copy(er_hbm.at[idxs[i][1]], erg, gsem)

    def _wait_gather(b, i):
        buf, erg, gsem, _ = rows[b]
        pltpu.make_async_copy(feat_hbm.at[idxs[i][0]], buf, gsem).wait()
        pltpu.make_async_copy(er_hbm.at[idxs[i][1]], erg, gsem).wait()

    def _issue_scatter(b, i):
        buf, _, _, ssem = rows[b]
        pltpu.async_copy(buf, acc_sh.at[idxs[i][1]], ssem, add=True)

    def _wait_scatter(b, i):
        buf, _, _, ssem = rows[b]
        pltpu.make_async_copy(buf, acc_sh.at[idxs[i][1]], ssem).wait()

    def _process(b):
        buf, erg, _, _ = rows[b]

        @plsc.parallel_loop(0, K // 16, unroll=2)
        def _ee(i):
            grp = lax.iota(jnp.int32, 16) + i * 16
            elg = plsc.load_gather(buf, [grp, jnp.full((16,), D + 1, jnp.int32)])
            e = elg + erg[pl.ds(i * 16, 16)]
            e = jnp.where(e > 0.0, e, NEG * e) - shift
            ee16 = jnp.exp(e)
            ee_v[pl.ds(i * 16, 16)] = ee16
            plsc.store_scatter(buf, [grp, jnp.full((16,), D, jnp.int32)], ee16)

        @plsc.parallel_loop(0, K, unroll=4)
        def _row(i):
            sc = plsc.load_gather(ee_v, [jnp.full((16,), i, jnp.int32)])
            for j in range(D // 16):
                buf[i, pl.ds(j * 16, 16)] = buf[i, pl.ds(j * 16, 16)] * sc

    def _step(jj, k):
        b = k % NBUF
        b1 = (k + 1) % NBUF
        i = k % NIDX
        i1 = (k + 1) % NIDX
        i2 = (k + 2) % NIDX

        @pl.when(jj >= 2)
        def _retire():
            _wait_scatter(b1, i2)

        @pl.when(jj + 1 < NCH)
        def _pref_gather():
            _wait_idx(i1)
            _issue_gather(b1, i1)

        _wait_gather(b, i)
        _process(b)
        _issue_scatter(b, i)

        @pl.when(jj + 2 < NCH)
        def _pref_idx():
            _issue_idx(jj + 2, i2)

    _issue_idx(0, 0)
    _issue_idx(1, 1)
    _wait_idx(0)
    _issue_gather(0, 0)

    def _span(p, carry):
        for k in range(SPAN):
            jj = p * SPAN + k

            @pl.when(jj < NCH)
            def _guarded():
                _step(jj, k)

        return carry

    lax.fori_loop(0, (NCH + SPAN - 1) // SPAN, _span, 0)
    _wait_scatter((NCH - 2) % NBUF, (NCH - 2) % NIDX)
    _wait_scatter((NCH - 1) % NBUF, (NCH - 1) % NIDX)
    plsc.subcore_barrier()

    r0 = s * ROWS_PER_TILE
    pltpu.sync_copy(acc_sh.at[pl.ds(r0, ROWS_PER_TILE)],
                    out_hbm.at[pl.ds(c * NP + r0, ROWS_PER_TILE)])


def kernel(x, edge_index, W1, al1, ar1, b1, W2, al2, ar2, b2):
    src = edge_index[0]
    dst = edge_index[1]
    xp = jnp.pad(x, ((0, NP - N), (0, 0)))
    feat1, er1, sh1 = _proj(xp, W1, al1, ar1)
    acc1 = _sc_edge(feat1, er1.reshape(NP), sh1.reshape(16), src, dst)
    feat2, er2, sh2 = _norm_proj(acc1, b1.reshape(1, D), W2, al2, ar2)
    acc2 = _sc_edge(feat2, er2.reshape(NP), sh2.reshape(16), src, dst)
    out = _final(acc2, b2.reshape(1, D))
    return out[:N]
